# baseline (speedup 1.0000x reference)
import functools

import numpy as np
import jax
import jax.numpy as jnp
from jax import lax
from jax.experimental import pallas as pl
from jax.experimental.pallas import tpu as pltpu
from jax.experimental.pallas import tpu_sc as plsc

_D = 128
_K = 29
_S = 32
_RPC = 4
_CSZ = _S * _RPC
_NG = _D // 32
_W = _D // 2

_PERM = np.zeros(_D, dtype=np.int32)
for _g in range(_NG):
    for _j in range(16):
        _PERM[_g * 32 + _j] = _g * 16 + _j
        _PERM[_g * 32 + 16 + _j] = 64 + _g * 16 + _j


def _embed_sum_sc(idx2d, emb_i32, nw, ch):
    rpt = ch * _RPC
    info = plsc.get_sparse_core_info()
    num_cores = info.num_cores
    mesh = plsc.VectorSubcoreMesh(core_axis_name="c", subcore_axis_name="s")

    @functools.partial(
        pl.kernel,
        mesh=mesh,
        out_type=jax.ShapeDtypeStruct((nw * rpt, _D), jnp.float32),
        compiler_params=pltpu.CompilerParams(use_tc_tiling_on_sc=False),
        scratch_types=[
            pltpu.VMEM((rpt, _S), jnp.int32),
            pltpu.VMEM((_S, _W), jnp.int32),
            pltpu.VMEM((_S, _W), jnp.int32),
            pltpu.VMEM((rpt, _D), jnp.float32),
            pltpu.VMEM_SHARED((7424, _W), jnp.int32),
            pltpu.SemaphoreType.DMA,
            pltpu.SemaphoreType.DMA,
        ],
    )
    def k(idx_hbm, emb_hbm, out_hbm, idx_v, buf0, buf1, out_v, tbl_s,
          sem0, sem1):
        wid = lax.axis_index("s") * num_cores + lax.axis_index("c")
        sid = lax.axis_index("s")


        @pl.when(sid == 0)
        def _():
            pltpu.sync_copy(emb_hbm, tbl_s)

        pltpu.sync_copy(idx_hbm.at[pl.ds(wid * rpt, rpt)], idx_v)
        plsc.subcore_barrier()

        sh16 = jnp.full((16,), 16, dtype=jnp.int32)

        def accum(buf, row):
            acc_lo = [None] * _NG
            acc_hi = [None] * _NG
            for kk in range(_K):
                for g in range(_NG):
                    w = buf[kk, pl.ds(g * 16, 16)]
                    lo = lax.bitcast_convert_type(
                        lax.shift_left(w, sh16), jnp.float32)
                    hi = lax.bitcast_convert_type(w, jnp.float32)
                    if kk == 0:
                        acc_lo[g], acc_hi[g] = lo, hi
                    else:
                        acc_lo[g] = acc_lo[g] + lo
                        acc_hi[g] = acc_hi[g] + hi
            for g in range(_NG):
                out_v[row, pl.ds(g * 32, 16)] = acc_lo[g]
                out_v[row, pl.ds(g * 32 + 16, 16)] = acc_hi[g]

        pltpu.async_copy(tbl_s.at[idx_v.at[0]], buf0, sem0)

        def body(j2, carry):
            c0 = 2 * j2
            c1 = c0 + 1
            pltpu.async_copy(
                tbl_s.at[idx_v.at[c1]], buf1, sem1)
            pltpu.make_async_copy(
                tbl_s.at[idx_v.at[c0]], buf0, sem0).wait()
            accum(buf0, c0)

            @pl.when(c0 + 2 < rpt)
            def _():
                pltpu.async_copy(tbl_s.at[idx_v.at[c0 + 2]], buf0, sem0)

            pltpu.make_async_copy(
                tbl_s.at[idx_v.at[c1]], buf1, sem1).wait()
            accum(buf1, c1)
            return carry

        lax.fori_loop(0, rpt // 2, body, 0)
        pltpu.sync_copy(out_v, out_hbm.at[pl.ds(wid * rpt, rpt)])

    return k(idx2d, emb_i32)


def _mlp_tc(h, W2pt, b2, W3at, W3bt, b3, W4a, W4b):
    B = h.shape[0]
    blk = 2048
    nb = B // blk

    def dot(a, w):
        return lax.dot_general(
            a, w, (((1,), (0,)), ((), ())),
            preferred_element_type=jnp.float32,
        )

    def dot_t1(a, w):
        return lax.dot_general(
            a, w, (((1,), (1,)), ((), ())),
            preferred_element_type=jnp.float32,
        )

    def body(h_ref, w2_ref, b2_ref, w3a_ref, w3b_ref, b3_ref, w4a_ref,
             w4b_ref, o_ref):
        a = jnp.maximum(h_ref[...], 0.0)
        t = dot(a, w2_ref[...]) + b2_ref[...]
        p = jnp.maximum(t, 0.0)
        m = jnp.maximum(-t, 0.0)
        u = dot(p, w3a_ref[...]) + dot(m, w3b_ref[...]) + b3_ref[...]
        p2 = jnp.maximum(u, 0.0)
        m2 = jnp.maximum(-u, 0.0)
        res = dot_t1(w4a_ref[...], p2) + dot_t1(w4b_ref[...], m2)
        o_ref[...] = res.reshape(1, 1, blk)

    return pl.pallas_call(
        body,
        grid=(nb,),
        in_specs=[
            pl.BlockSpec((blk, _D), lambda i: (i, 0)),
            pl.BlockSpec((_D, 32), lambda i: (0, 0)),
            pl.BlockSpec((1, 32), lambda i: (0, 0)),
            pl.BlockSpec((32, 32), lambda i: (0, 0)),
            pl.BlockSpec((32, 32), lambda i: (0, 0)),
            pl.BlockSpec((1, 32), lambda i: (0, 0)),
            pl.BlockSpec((1, 32), lambda i: (0, 0)),
            pl.BlockSpec((1, 32), lambda i: (0, 0)),
        ],
        out_specs=pl.BlockSpec((1, 1, blk), lambda i: (i, 0, 0)),
        out_shape=jax.ShapeDtypeStruct((nb, 1, blk), jnp.float32),
    )(h, W2pt, b2.reshape(1, 32), W3at, W3bt, b3.reshape(1, 32),
      W4a, W4b)


def kernel(x, emb, W2, b2, W3, b3, W4):
    B = x.shape[0]
    info = plsc.get_sparse_core_info()
    nw = info.num_cores * info.num_subcores
    ch = B // (nw * _RPC)
    idx2d = x.astype(jnp.int32)
    u = lax.bitcast_convert_type(emb.astype(jnp.bfloat16), jnp.uint16)
    w = (u[:, :_W].astype(jnp.uint32)
         | lax.shift_left(u[:, _W:].astype(jnp.uint32), jnp.uint32(16)))
    emb_i32 = lax.bitcast_convert_type(w, jnp.int32)
    h = _embed_sum_sc(idx2d, emb_i32, nw, ch)
    W2pt = W2[:, _PERM].T
    W3at = W3[:, :32].T
    W3bt = W3[:, 32:].T
    W4a = W4[:, :32]
    W4b = W4[:, 32:]
    out = _mlp_tc(h, W2pt, b2, W3at, W3bt, b3, W4a, W4b)
    return out.reshape(B, 1)


# speedup vs baseline: 1.1858x; 1.1858x over previous
import functools

import numpy as np
import jax
import jax.numpy as jnp
from jax import lax
from jax.experimental import pallas as pl
from jax.experimental.pallas import tpu as pltpu
from jax.experimental.pallas import tpu_sc as plsc

_D = 128
_K = 29
_S = 32
_RPC = 4
_CSZ = _S * _RPC
_NG = _D // 32
_W = _D // 2

_PERM = np.zeros(_D, dtype=np.int32)
for _g in range(_NG):
    for _j in range(16):
        _PERM[_g * 32 + _j] = _g * 16 + _j
        _PERM[_g * 32 + 16 + _j] = 64 + _g * 16 + _j


def _embed_sum_sc(idx_flat, emb_i32, nw, ch):
    rpt = ch * _RPC
    info = plsc.get_sparse_core_info()
    num_cores = info.num_cores
    mesh = plsc.VectorSubcoreMesh(core_axis_name="c", subcore_axis_name="s")

    @functools.partial(
        pl.kernel,
        mesh=mesh,
        out_type=jax.ShapeDtypeStruct((nw * rpt, _D), jnp.float32),
        compiler_params=pltpu.CompilerParams(use_tc_tiling_on_sc=False),
        scratch_types=[
            pltpu.VMEM((ch * _CSZ,), jnp.int32),
            pltpu.VMEM((_CSZ, _W), jnp.int32),
            pltpu.VMEM((_CSZ, _W), jnp.int32),
            pltpu.VMEM((rpt, _D), jnp.float32),
            pltpu.VMEM_SHARED((7424, _W), jnp.int32),
            pltpu.SemaphoreType.DMA,
            pltpu.SemaphoreType.DMA,
        ],
    )
    def k(idx_hbm, emb_hbm, out_hbm, idx_v, buf0, buf1, out_v, tbl_s,
          sem0, sem1):
        wid = lax.axis_index("s") * num_cores + lax.axis_index("c")
        sid = lax.axis_index("s")


        @pl.when(sid == 0)
        def _():
            pltpu.sync_copy(emb_hbm, tbl_s)

        pltpu.sync_copy(idx_hbm.at[pl.ds(wid * ch * _CSZ, ch * _CSZ)], idx_v)
        plsc.subcore_barrier()

        sh16 = jnp.full((16,), 16, dtype=jnp.int32)

        def accum(buf, c):
            for r in range(_RPC):
                acc_lo = [None] * _NG
                acc_hi = [None] * _NG
                for kk in range(_K):
                    for g in range(_NG):
                        w = buf[r * _S + kk, pl.ds(g * 16, 16)]
                        lo = lax.bitcast_convert_type(
                            lax.shift_left(w, sh16), jnp.float32)
                        hi = lax.bitcast_convert_type(w, jnp.float32)
                        if kk == 0:
                            acc_lo[g], acc_hi[g] = lo, hi
                        else:
                            acc_lo[g] = acc_lo[g] + lo
                            acc_hi[g] = acc_hi[g] + hi
                row = c * _RPC + r
                for g in range(_NG):
                    out_v[row, pl.ds(g * 32, 16)] = acc_lo[g]
                    out_v[row, pl.ds(g * 32 + 16, 16)] = acc_hi[g]

        pltpu.async_copy(tbl_s.at[idx_v.at[pl.ds(0, _CSZ)]], buf0, sem0)

        def body(j2, carry):
            c0 = 2 * j2
            c1 = c0 + 1
            pltpu.async_copy(
                tbl_s.at[idx_v.at[pl.ds(c1 * _CSZ, _CSZ)]], buf1, sem1)
            pltpu.make_async_copy(
                tbl_s.at[idx_v.at[pl.ds(c0 * _CSZ, _CSZ)]], buf0, sem0).wait()
            accum(buf0, c0)

            @pl.when(c0 + 2 < ch)
            def _():
                pltpu.async_copy(
                    tbl_s.at[idx_v.at[pl.ds((c0 + 2) * _CSZ, _CSZ)]], buf0, sem0)

            pltpu.make_async_copy(
                tbl_s.at[idx_v.at[pl.ds(c1 * _CSZ, _CSZ)]], buf1, sem1).wait()
            accum(buf1, c1)
            return carry

        lax.fori_loop(0, ch // 2, body, 0)
        pltpu.sync_copy(out_v, out_hbm.at[pl.ds(wid * rpt, rpt)])

    return k(idx_flat, emb_i32)


def _mlp_tc(h, W2pt, b2, W3at, W3bt, b3, W4a, W4b):
    B = h.shape[0]
    blk = 2048
    nb = B // blk

    def dot(a, w):
        return lax.dot_general(
            a, w, (((1,), (0,)), ((), ())),
            preferred_element_type=jnp.float32,
        )

    def dot_t1(a, w):
        return lax.dot_general(
            a, w, (((1,), (1,)), ((), ())),
            preferred_element_type=jnp.float32,
        )

    def body(h_ref, w2_ref, b2_ref, w3a_ref, w3b_ref, b3_ref, w4a_ref,
             w4b_ref, o_ref):
        a = jnp.maximum(h_ref[...], 0.0)
        t = dot(a, w2_ref[...]) + b2_ref[...]
        p = jnp.maximum(t, 0.0)
        m = jnp.maximum(-t, 0.0)
        u = dot(p, w3a_ref[...]) + dot(m, w3b_ref[...]) + b3_ref[...]
        p2 = jnp.maximum(u, 0.0)
        m2 = jnp.maximum(-u, 0.0)
        res = dot_t1(w4a_ref[...], p2) + dot_t1(w4b_ref[...], m2)
        o_ref[...] = res.reshape(1, 1, blk)

    return pl.pallas_call(
        body,
        grid=(nb,),
        in_specs=[
            pl.BlockSpec((blk, _D), lambda i: (i, 0)),
            pl.BlockSpec((_D, 32), lambda i: (0, 0)),
            pl.BlockSpec((1, 32), lambda i: (0, 0)),
            pl.BlockSpec((32, 32), lambda i: (0, 0)),
            pl.BlockSpec((32, 32), lambda i: (0, 0)),
            pl.BlockSpec((1, 32), lambda i: (0, 0)),
            pl.BlockSpec((1, 32), lambda i: (0, 0)),
            pl.BlockSpec((1, 32), lambda i: (0, 0)),
        ],
        out_specs=pl.BlockSpec((1, 1, blk), lambda i: (i, 0, 0)),
        out_shape=jax.ShapeDtypeStruct((nb, 1, blk), jnp.float32),
    )(h, W2pt, b2.reshape(1, 32), W3at, W3bt, b3.reshape(1, 32),
      W4a, W4b)


def kernel(x, emb, W2, b2, W3, b3, W4):
    B = x.shape[0]
    info = plsc.get_sparse_core_info()
    nw = info.num_cores * info.num_subcores
    ch = B // (nw * _RPC)
    idx_flat = x.astype(jnp.int32).reshape(-1)
    u = lax.bitcast_convert_type(emb.astype(jnp.bfloat16), jnp.uint16)
    w = (u[:, :_W].astype(jnp.uint32)
         | lax.shift_left(u[:, _W:].astype(jnp.uint32), jnp.uint32(16)))
    emb_i32 = lax.bitcast_convert_type(w, jnp.int32)
    h = _embed_sum_sc(idx_flat, emb_i32, nw, ch)
    W2pt = W2[:, _PERM].T
    W3at = W3[:, :32].T
    W3bt = W3[:, 32:].T
    W4a = W4[:, :32]
    W4b = W4[:, 32:]
    out = _mlp_tc(h, W2pt, b2, W3at, W3bt, b3, W4a, W4b)
    return out.reshape(B, 1)
